# baseline (speedup 1.0000x reference)
import jax
import jax.numpy as jnp
from jax import lax
from jax.experimental import pallas as pl
from jax.experimental.pallas import tpu as pltpu
from jax.experimental.pallas import tpu_sc as plsc

N = 10000
E = 320000
NFEAT = 128
NHID = 64
NHEAD = 4
NCLASS = 16
ALPHA = 0.2

G = 128
NCHUNK = 2560
E_PAD = NCHUNK * G
SUP = 16
WS1 = 80
WS2 = 32
NSC = 2
NTILE = 16
NRC = 79
TAIL = N - (NRC - 1) * G
TB = 1000
NBUFG = 4
NBUFS = 2
LA = 3
f32 = jnp.float32
bf16 = jnp.bfloat16
i32 = jnp.int32


def _acc_chunks(s, fn_full, fn_tail):
    for k in range(5):
        j = s + NTILE * k
        off = pl.multiple_of(j * G, G)

        @pl.when(j < NRC - 1)
        def _():
            fn_full(off)

        @pl.when(j == NRC - 1)
        def _():
            fn_tail(off)


def _zero_buf0(buf, width):
    zv = jnp.zeros((16,), f32)
    offs = [k * 16 for k in range(width // 16)]
    if width % 16:
        offs.append(width - 16)

    @pl.loop(0, G)
    def _(r):
        for o in offs:
            buf[0, r, pl.ds(o, 16)] = zv


def _edge_ring(nch, cbase, coff, tbl_hbm, src_hbm, dst_hbm,
               srcs_v, dsts_v, gbuf, sbuf, gsem, ssem, acc_s, attend_scale):
    def stage_sup(sup):
        db = lax.rem(sup, 2)
        gb = cbase + sup * SUP
        pltpu.sync_copy(src_hbm.at[pl.ds(gb, SUP)], srcs_v.at[db])
        pltpu.sync_copy(dst_hbm.at[pl.ds(gb, SUP)], dsts_v.at[db])

        if coff is not None:
            @pl.loop(0, SUP)
            def _(t):
                for g in range(8):
                    dsts_v[db, t, pl.ds(g * 16, 16)] = (
                        dsts_v[db, t, pl.ds(g * 16, 16)] + coff)

    def gather_desc(j, b):
        db = lax.rem(lax.div(j, SUP), 2)
        t = lax.rem(j, SUP)
        return pltpu.make_async_copy(
            tbl_hbm.at[dsts_v.at[db, t]], gbuf.at[b], gsem.at[b])

    def scatter_desc(j, b):
        db = lax.rem(lax.div(j, SUP), 2)
        t = lax.rem(j, SUP)
        return pltpu.make_async_copy(
            sbuf.at[b], acc_s.at[srcs_v.at[db, t]], ssem.at[b])

    stage_sup(0)
    for j0 in range(LA):
        gather_desc(j0, j0 % NBUFG).start()

    @pl.loop(0, nch)
    def _(j):
        @pl.when(jnp.logical_and(lax.rem(j + LA, SUP) == 0, j + LA < nch))
        def _():
            stage_sup(lax.div(j + LA, SUP))

        @pl.when(j + LA < nch)
        def _():
            gather_desc(j + LA, lax.rem(j + LA, NBUFG)).start()

        gather_desc(j, lax.rem(j, NBUFG)).wait()

        @pl.when(j >= NBUFS)
        def _():
            scatter_desc(j - NBUFS, lax.rem(j, NBUFS)).wait()

        t = lax.rem(j, SUP)
        db = lax.rem(lax.div(j, SUP), 2)
        attend_scale(lax.rem(j, NBUFG), lax.rem(j, NBUFS), db, t, cbase + j)
        scatter_desc(j, lax.rem(j, NBUFS)).start(add=True)

    for jd in range(nch - NBUFS, nch):
        scatter_desc(jd, jd % NBUFS).wait()


def _tc_prep1(feat_ref, w1_ref, acat_ref, h_ref, al_ref):
    x = feat_ref[...]
    h = jnp.dot(x, w1_ref[...], preferred_element_type=f32)
    al = jnp.dot(h, acat_ref[...], preferred_element_type=f32)
    al_ref[...] = al
    h_ref[...] = h


def _sc_l1(htbl_hbm, alt_hbm, src_hbm, dst_hbm, acc_hbm,
           as_v, ad_v, srcs_v, dsts_v, gbuf, sbuf, gsem, ssem, acc_s):
    c = lax.axis_index("c")
    s = lax.axis_index("s")
    iota16 = lax.broadcasted_iota(i32, (16,), 0)
    c64 = jnp.full((16,), NHID, i32)
    cbase = s * (NCHUNK // NTILE)

    for hh in range(2):
        hd = 2 * c + hh
        coff = hd * N
        pltpu.sync_copy(alt_hbm.at[hd], as_v)
        pltpu.sync_copy(alt_hbm.at[NHEAD + hd], ad_v)
        _zero_buf0(sbuf, WS1)
        _acc_chunks(
            s,
            lambda off: pltpu.sync_copy(sbuf.at[0], acc_s.at[pl.ds(off, G)]),
            lambda off: pltpu.sync_copy(sbuf.at[0, 0:TAIL],
                                        acc_s.at[pl.ds(off, TAIL)]))
        plsc.subcore_barrier()

        def attend_scale(gb, sb, db, t, jg):
            ebase = jg * G
            for g in range(8):
                e16 = iota16 + (g * 16)
                idx_s = srcs_v[db, t, pl.ds(g * 16, 16)]
                idx_d = dsts_v[db, t, pl.ds(g * 16, 16)] - coff
                a_s = plsc.load_gather(as_v, [idx_s])
                a_d = plsc.load_gather(ad_v, [idx_d])
                l = a_s + a_d
                w = jnp.exp(jnp.where(l > 0, l, ALPHA * l))
                w = jnp.where((ebase + g * 16 + iota16) < E, w, 0.0)
                plsc.store_scatter(sbuf.at[sb], [e16, c64], w)

            @pl.loop(0, G, unroll=4)
            def _(e):
                ef = jnp.full((16,), e, i32)
                w = plsc.load_gather(sbuf.at[sb], [ef, c64])
                for k in range(2):
                    x2 = gbuf[gb, e, pl.ds(k * 32, 32)]
                    lo, hi = plsc.unpack(x2, format=plsc.PackFormat.INTERLEAVED)
                    sbuf[sb, e, pl.ds(k * 32, 16)] = lo * w
                    sbuf[sb, e, pl.ds(k * 32 + 16, 16)] = hi * w

        _edge_ring(NCHUNK // NTILE, cbase, coff, htbl_hbm, src_hbm, dst_hbm,
                   srcs_v, dsts_v, gbuf, sbuf, gsem, ssem, acc_s, attend_scale)
        plsc.subcore_barrier()
        _acc_chunks(
            s,
            lambda off: pltpu.sync_copy(acc_s.at[pl.ds(off, G)],
                                        acc_hbm.at[hd, pl.ds(off, G)]),
            lambda off: pltpu.sync_copy(acc_s.at[pl.ds(off, TAIL)],
                                        acc_hbm.at[hd, pl.ds(off, TAIL)]))
        if hh == 0:
            plsc.subcore_barrier()


def _tc_prep2(acc_ref, w2_ref, b2_ref, a2_ref, htbl2_ref, al2_ref):
    xs = [acc_ref[k][:, 0:NHID] / acc_ref[k][:, NHID:NHID + 1]
          for k in range(NHEAD)]
    x = jnp.concatenate(xs, axis=1)
    x = jnp.where(x > 0, x, jnp.exp(jnp.minimum(x, 0.0)) - 1.0)
    h2 = jnp.dot(x, w2_ref[...], preferred_element_type=f32) + b2_ref[...]
    al = jnp.dot(h2, a2_ref[...], preferred_element_type=f32)
    htbl2_ref[...] = h2
    al2_ref[...] = jnp.concatenate([al, jnp.zeros((TB, 6), f32)], 1)


def _sc_l2(htbl2_hbm, al2t_hbm, src_hbm, dst_hbm, acc_hbm,
           as_v, ad_v, srcs_v, dsts_v, gbuf, sbuf, gsem, ssem, acc_s):
    c = lax.axis_index("c")
    s = lax.axis_index("s")
    wid = c * NTILE + s
    iota16 = lax.broadcasted_iota(i32, (16,), 0)
    c16 = jnp.full((16,), NCLASS, i32)
    pltpu.sync_copy(al2t_hbm.at[0], as_v)
    pltpu.sync_copy(al2t_hbm.at[1], ad_v)
    _zero_buf0(sbuf, WS2)
    _acc_chunks(
        s,
        lambda off: pltpu.sync_copy(sbuf.at[0], acc_s.at[pl.ds(off, G)]),
        lambda off: pltpu.sync_copy(sbuf.at[0, 0:TAIL],
                                    acc_s.at[pl.ds(off, TAIL)]))
    plsc.subcore_barrier()

    nch = NCHUNK // (NSC * NTILE)
    cbase = wid * nch

    def attend_scale(gb, sb, db, t, jg):
        ebase = jg * G
        for g in range(8):
            e16 = iota16 + (g * 16)
            idx_s = srcs_v[db, t, pl.ds(g * 16, 16)]
            idx_d = dsts_v[db, t, pl.ds(g * 16, 16)]
            a_s = plsc.load_gather(as_v, [idx_s])
            a_d = plsc.load_gather(ad_v, [idx_d])
            l = a_s + a_d
            w = jnp.exp(jnp.where(l > 0, l, ALPHA * l))
            w = jnp.where((ebase + g * 16 + iota16) < E, w, 0.0)
            plsc.store_scatter(sbuf.at[sb], [e16, c16], w)

        @pl.loop(0, G, unroll=8)
        def _(e):
            ef = jnp.full((16,), e, i32)
            w = plsc.load_gather(sbuf.at[sb], [ef, c16])
            sbuf[sb, e, pl.ds(0, 16)] = gbuf[gb, e, pl.ds(0, 16)] * w

    _edge_ring(nch, cbase, None, htbl2_hbm, src_hbm, dst_hbm,
               srcs_v, dsts_v, gbuf, sbuf, gsem, ssem, acc_s, attend_scale)
    plsc.subcore_barrier()
    _acc_chunks(
        s,
        lambda off: pltpu.sync_copy(acc_s.at[pl.ds(off, G)],
                                    acc_hbm.at[c, pl.ds(off, G)]),
        lambda off: pltpu.sync_copy(acc_s.at[pl.ds(off, TAIL)],
                                    acc_hbm.at[c, pl.ds(off, TAIL)]))


def _tc_final(acc_ref, out_ref):
    t = acc_ref[0] + acc_ref[1]
    h = t[:, 0:NCLASS] / t[:, NCLASS:NCLASS + 1]
    z = h - jnp.max(h, axis=1, keepdims=True)
    out_ref[...] = z - jnp.log(jnp.sum(jnp.exp(z), axis=1, keepdims=True))


def _sc_mesh():
    return plsc.VectorSubcoreMesh(core_axis_name="c", subcore_axis_name="s",
                                  num_cores=NSC, num_subcores=NTILE)


_SC_PARAMS = pltpu.CompilerParams(needs_layout_passes=False,
                                  use_tc_tiling_on_sc=False)


def kernel(features, edge_list, W_heads, b_heads, a_heads, W_out, b_out, a_out):
    W1 = W_heads.reshape(NHEAD * NHID, NFEAT).T
    asrc = a_heads[:, 0, :NHID]
    adst = a_heads[:, 0, NHID:]
    eye = jnp.eye(NHEAD, dtype=f32)
    A_as = (eye[:, None, :] * asrc[:, :, None]).reshape(NHEAD * NHID, NHEAD)
    A_ad = (eye[:, None, :] * adst[:, :, None]).reshape(NHEAD * NHID, NHEAD)
    Acat = jnp.concatenate([A_as, A_ad], axis=1)
    pad = E_PAD - E
    src2d = jnp.pad(edge_list[0], (0, pad)).reshape(NCHUNK, G)
    dst2d = jnp.pad(edge_list[1], (0, pad)).reshape(NCHUNK, G)

    h_all, al_nm = pl.pallas_call(
        _tc_prep1,
        grid=(N // TB,),
        in_specs=[pl.BlockSpec((TB, NFEAT), lambda i: (i, 0)),
                  pl.BlockSpec((NFEAT, 256), lambda i: (0, 0)),
                  pl.BlockSpec((256, 8), lambda i: (0, 0))],
        out_specs=[pl.BlockSpec((TB, 256), lambda i: (i, 0)),
                   pl.BlockSpec((TB, 8), lambda i: (i, 0))],
        out_shape=[jax.ShapeDtypeStruct((N, 256), f32),
                   jax.ShapeDtypeStruct((N, 8), f32)],
    )(features, W1, Acat)
    htbl2d = (h_all.reshape(N, NHEAD, 2, 2, 16)
              .transpose(1, 0, 2, 4, 3)
              .reshape(NHEAD * N, NHID).astype(bf16))
    alt = al_nm.T

    acc1 = pl.kernel(
        _sc_l1,
        out_type=jax.ShapeDtypeStruct((NHEAD, N, WS1), f32),
        mesh=_sc_mesh(),
        compiler_params=_SC_PARAMS,
        scratch_types=[
            pltpu.VMEM((N,), f32),
            pltpu.VMEM((N,), f32),
            pltpu.VMEM((2, SUP, G), i32),
            pltpu.VMEM((2, SUP, G), i32),
            pltpu.VMEM((NBUFG, G, NHID), bf16),
            pltpu.VMEM((NBUFS, G, WS1), f32),
            pltpu.SemaphoreType.DMA((NBUFG,)),
            pltpu.SemaphoreType.DMA((NBUFS,)),
            pltpu.VMEM_SHARED((N, WS1), f32),
        ],
    )(htbl2d, alt, src2d, dst2d)

    W2 = W_out.T
    a2cat = jnp.stack([a_out[0, :NCLASS], a_out[0, NCLASS:]], axis=1)
    htbl2, al2_nm = pl.pallas_call(
        _tc_prep2,
        grid=(N // TB,),
        in_specs=[pl.BlockSpec((NHEAD, TB, WS1), lambda i: (0, i, 0)),
                  pl.BlockSpec((256, NCLASS), lambda i: (0, 0)),
                  pl.BlockSpec((1, NCLASS), lambda i: (0, 0)),
                  pl.BlockSpec((NCLASS, 2), lambda i: (0, 0))],
        out_specs=[pl.BlockSpec((TB, NCLASS), lambda i: (i, 0)),
                   pl.BlockSpec((TB, 8), lambda i: (i, 0))],
        out_shape=[jax.ShapeDtypeStruct((N, NCLASS), f32),
                   jax.ShapeDtypeStruct((N, 8), f32)],
    )(acc1, W2, b_out.reshape(1, NCLASS), a2cat)
    al2t = al2_nm[:, 0:2].T

    acc2 = pl.kernel(
        _sc_l2,
        out_type=jax.ShapeDtypeStruct((NSC, N, WS2), f32),
        mesh=_sc_mesh(),
        compiler_params=_SC_PARAMS,
        scratch_types=[
            pltpu.VMEM((N,), f32),
            pltpu.VMEM((N,), f32),
            pltpu.VMEM((2, SUP, G), i32),
            pltpu.VMEM((2, SUP, G), i32),
            pltpu.VMEM((NBUFG, G, NCLASS), f32),
            pltpu.VMEM((NBUFS, G, WS2), f32),
            pltpu.SemaphoreType.DMA((NBUFG,)),
            pltpu.SemaphoreType.DMA((NBUFS,)),
            pltpu.VMEM_SHARED((N, WS2), f32),
        ],
    )(htbl2, al2t, src2d, dst2d)

    return pl.pallas_call(
        _tc_final,
        grid=(N // TB,),
        in_specs=[pl.BlockSpec((2, TB, WS2), lambda i: (0, i, 0))],
        out_specs=pl.BlockSpec((TB, NCLASS), lambda i: (i, 0)),
        out_shape=jax.ShapeDtypeStruct((N, NCLASS), f32),
    )(acc2)
